# baseline (speedup 1.0000x reference)
import functools

import jax
import jax.numpy as jnp
from jax import lax
from jax.experimental import pallas as pl
from jax.experimental.pallas import tpu as pltpu
from jax.experimental.pallas import tpu_sc as plsc

B = 16384
D = 128
NW = 32
NSPLIT = 2
BS = B // NSPLIT
BPW = BS // NW
CHUNK = 128
NCH = BPW // CHUNK
BLK = 8192


def _sc_gather_body(split, uidx_hbm, ridx_hbm, utab_hbm, rtab_hbm,
                    uout_hbm, rout_hbm, uidx_v, ridx_v, urows, rrows,
                    gsem_u, gsem_r, wsem):
    wid = lax.axis_index("s") * 2 + lax.axis_index("c")
    base = split * BS + wid * BPW
    iu = pltpu.async_copy(uidx_hbm.at[pl.ds(base, BPW)], uidx_v, wsem)
    ir = pltpu.async_copy(ridx_hbm.at[pl.ds(base, BPW)], ridx_v, wsem)
    iu.wait()
    ir.wait()
    gu = [pltpu.async_copy(utab_hbm.at[uidx_v.at[pl.ds(ch * CHUNK, CHUNK)]],
                           urows.at[pl.ds(ch * CHUNK, CHUNK)], gsem_u)
          for ch in range(NCH)]
    gr = [pltpu.async_copy(rtab_hbm.at[ridx_v.at[pl.ds(ch * CHUNK, CHUNK)]],
                           rrows.at[pl.ds(ch * CHUNK, CHUNK)], gsem_r)
          for ch in range(NCH)]
    for c in gu:
        c.wait()
    wu = pltpu.async_copy(urows, uout_hbm.at[pl.ds(wid * BPW, BPW)], wsem)
    for c in gr:
        c.wait()
    wr = pltpu.async_copy(rrows, rout_hbm.at[pl.ds(wid * BPW, BPW)], wsem)
    wu.wait()
    wr.wait()


@functools.cache
def _sc_gather(split):
    return pl.kernel(
        functools.partial(_sc_gather_body, split),
        out_type=[jax.ShapeDtypeStruct((BS, D), jnp.float32),
                  jax.ShapeDtypeStruct((BS, D), jnp.float32)],
        mesh=plsc.VectorSubcoreMesh(core_axis_name="c", subcore_axis_name="s"),
        scratch_types=[
            pltpu.VMEM((BPW,), jnp.int32),
            pltpu.VMEM((BPW,), jnp.int32),
            pltpu.VMEM((BPW, D), jnp.float32),
            pltpu.VMEM((BPW, D), jnp.float32),
            pltpu.SemaphoreType.DMA,
            pltpu.SemaphoreType.DMA,
            pltpu.SemaphoreType.DMA,
        ],
    )


def _mlp_body(re_ref, ue_ref, w1_ref, b1_ref, w2_ref, b2_ref,
              w3_ref, b3_ref, *rest):
    out_ref = rest[-1]
    w1 = w1_ref[...].astype(jnp.bfloat16)
    reb = re_ref[...].astype(jnp.bfloat16)
    ueb = ue_ref[...].astype(jnp.bfloat16)
    r1 = jnp.dot(reb, w1[:D], preferred_element_type=jnp.float32)
    r1 += jnp.dot(ueb, w1[D:], preferred_element_type=jnp.float32)
    r1 = jnp.maximum(r1 + b1_ref[...], 0.0).astype(jnp.bfloat16)
    r2 = jnp.dot(r1, w2_ref[...].astype(jnp.bfloat16),
                 preferred_element_type=jnp.float32)
    r2 = jnp.maximum(r2 + b2_ref[...], 0.0)
    p = r2 * w3_ref[...]
    outs = []
    for c in range(BLK // 128):
        pc = p[c * 128:(c + 1) * 128, :]
        outs.append(jnp.sum(pc.T, axis=0, keepdims=True))
    out_ref[...] = jnp.concatenate(outs, axis=0) + b3_ref[...]


def _mlp(re, ue, w1, b1, w2, b2, w3, b3, s, prev=None):
    h1 = w1.shape[1]
    h2 = w2.shape[1]
    rpb = BLK // 128
    off = s * (BS // BLK)
    full = lambda shape: pl.BlockSpec(shape, lambda i: (0, 0))
    out_spec = pl.BlockSpec((rpb, 128), lambda i: (off + i, 0))
    in_specs = [
        pl.BlockSpec((BLK, D), lambda i: (i, 0)),
        pl.BlockSpec((BLK, D), lambda i: (i, 0)),
        full((2 * D, h1)), full((1, h1)),
        full((h1, h2)), full((1, h2)),
        full((1, h2)), full((1, 1)),
    ]
    args = [re, ue, w1, b1, w2, b2, w3, b3]
    kwargs = {}
    if prev is not None:
        in_specs.append(pl.BlockSpec((rpb, 128), lambda i: (off + i, 0)))
        args.append(prev)
        kwargs["input_output_aliases"] = {8: 0}
    return pl.pallas_call(
        _mlp_body,
        grid=(BS // BLK,),
        in_specs=in_specs,
        out_specs=out_spec,
        out_shape=jax.ShapeDtypeStruct((B // 128, 128), jnp.float32),
        **kwargs,
    )(*args)


def kernel(user, recipe, user_table, recipe_table, W1, b1, W2, b2, W3, b3):
    uidx = user.astype(jnp.int32)
    ridx = recipe.astype(jnp.int32)

    h1 = W1.shape[1]
    h2 = W2.shape[1]
    b1r = b1.reshape(1, h1)
    b2r = b2.reshape(1, h2)
    w3r = W3.reshape(1, h2)
    b3r = b3.reshape(1, 1)

    embs = [_sc_gather(s)(uidx, ridx, user_table, recipe_table)
            for s in range(NSPLIT)]
    out = None
    for s, (ue, re) in enumerate(embs):
        out = _mlp(re, ue, W1, b1r, W2, b2r, w3r, b3r, s, prev=out)
    return out.reshape(B)


# speedup vs baseline: 1.0215x; 1.0215x over previous
import functools

import jax
import jax.numpy as jnp
from jax import lax
from jax.experimental import pallas as pl
from jax.experimental.pallas import tpu as pltpu
from jax.experimental.pallas import tpu_sc as plsc

B = 16384
D = 128
NW = 32
NSPLIT = 2
BS = B // NSPLIT
BPW = BS // NW
CHUNK = 128
NCH = BPW // CHUNK
BLK = 4096


def _sc_gather_body(split, uidx_hbm, ridx_hbm, utab_hbm, rtab_hbm,
                    uout_hbm, rout_hbm, uidx_v, ridx_v, urows, rrows,
                    gsem_u, gsem_r, wsem):
    wid = lax.axis_index("s") * 2 + lax.axis_index("c")
    base = split * BS + wid * BPW
    iu = pltpu.async_copy(uidx_hbm.at[pl.ds(base, BPW)], uidx_v, wsem)
    ir = pltpu.async_copy(ridx_hbm.at[pl.ds(base, BPW)], ridx_v, wsem)
    iu.wait()
    ir.wait()
    gu = [pltpu.async_copy(utab_hbm.at[uidx_v.at[pl.ds(ch * CHUNK, CHUNK)]],
                           urows.at[pl.ds(ch * CHUNK, CHUNK)], gsem_u)
          for ch in range(NCH)]
    gr = [pltpu.async_copy(rtab_hbm.at[ridx_v.at[pl.ds(ch * CHUNK, CHUNK)]],
                           rrows.at[pl.ds(ch * CHUNK, CHUNK)], gsem_r)
          for ch in range(NCH)]
    for c in gu:
        c.wait()
    wu = pltpu.async_copy(urows, uout_hbm.at[pl.ds(wid * BPW, BPW)], wsem)
    for c in gr:
        c.wait()
    wr = pltpu.async_copy(rrows, rout_hbm.at[pl.ds(wid * BPW, BPW)], wsem)
    wu.wait()
    wr.wait()


@functools.cache
def _sc_gather(split):
    return pl.kernel(
        functools.partial(_sc_gather_body, split),
        out_type=[jax.ShapeDtypeStruct((BS, D), jnp.float32),
                  jax.ShapeDtypeStruct((BS, D), jnp.float32)],
        mesh=plsc.VectorSubcoreMesh(core_axis_name="c", subcore_axis_name="s"),
        scratch_types=[
            pltpu.VMEM((BPW,), jnp.int32),
            pltpu.VMEM((BPW,), jnp.int32),
            pltpu.VMEM((BPW, D), jnp.float32),
            pltpu.VMEM((BPW, D), jnp.float32),
            pltpu.SemaphoreType.DMA,
            pltpu.SemaphoreType.DMA,
            pltpu.SemaphoreType.DMA,
        ],
    )


def _mlp_body(re_ref, ue_ref, w1_ref, b1_ref, w2_ref, b2_ref,
              w3_ref, b3_ref, *rest):
    out_ref = rest[-1]
    w1 = w1_ref[...].astype(jnp.bfloat16)
    reb = re_ref[...].astype(jnp.bfloat16)
    ueb = ue_ref[...].astype(jnp.bfloat16)
    r1 = jnp.dot(reb, w1[:D], preferred_element_type=jnp.float32)
    r1 += jnp.dot(ueb, w1[D:], preferred_element_type=jnp.float32)
    r1 = jnp.maximum(r1 + b1_ref[...], 0.0).astype(jnp.bfloat16)
    r2 = jnp.dot(r1, w2_ref[...].astype(jnp.bfloat16),
                 preferred_element_type=jnp.float32)
    r2 = jnp.maximum(r2 + b2_ref[...], 0.0)
    p = r2 * w3_ref[...]
    outs = []
    for c in range(BLK // 128):
        pc = p[c * 128:(c + 1) * 128, :]
        outs.append(jnp.sum(pc.T, axis=0, keepdims=True))
    out_ref[...] = jnp.concatenate(outs, axis=0) + b3_ref[...]


def _mlp(re, ue, w1, b1, w2, b2, w3, b3, s, prev=None):
    h1 = w1.shape[1]
    h2 = w2.shape[1]
    rpb = BLK // 128
    off = s * (BS // BLK)
    full = lambda shape: pl.BlockSpec(shape, lambda i: (0, 0))
    out_spec = pl.BlockSpec((rpb, 128), lambda i: (off + i, 0))
    in_specs = [
        pl.BlockSpec((BLK, D), lambda i: (i, 0)),
        pl.BlockSpec((BLK, D), lambda i: (i, 0)),
        full((2 * D, h1)), full((1, h1)),
        full((h1, h2)), full((1, h2)),
        full((1, h2)), full((1, 1)),
    ]
    args = [re, ue, w1, b1, w2, b2, w3, b3]
    kwargs = {}
    if prev is not None:
        in_specs.append(pl.BlockSpec((rpb, 128), lambda i: (off + i, 0)))
        args.append(prev)
        kwargs["input_output_aliases"] = {8: 0}
    return pl.pallas_call(
        _mlp_body,
        grid=(BS // BLK,),
        in_specs=in_specs,
        out_specs=out_spec,
        out_shape=jax.ShapeDtypeStruct((B // 128, 128), jnp.float32),
        **kwargs,
    )(*args)


def kernel(user, recipe, user_table, recipe_table, W1, b1, W2, b2, W3, b3):
    uidx = user.astype(jnp.int32)
    ridx = recipe.astype(jnp.int32)

    h1 = W1.shape[1]
    h2 = W2.shape[1]
    b1r = b1.reshape(1, h1)
    b2r = b2.reshape(1, h2)
    w3r = W3.reshape(1, h2)
    b3r = b3.reshape(1, 1)

    embs = [_sc_gather(s)(uidx, ridx, user_table, recipe_table)
            for s in range(NSPLIT)]
    out = None
    for s, (ue, re) in enumerate(embs):
        out = _mlp(re, ue, W1, b1r, W2, b2r, w3r, b3r, s, prev=out)
    return out.reshape(B)
